# baseline (speedup 1.0000x reference)
import jax
import jax.numpy as jnp
from jax import lax
from jax.experimental import pallas as pl
from jax.experimental.pallas import tpu as pltpu
from jax.experimental.pallas import tpu_sc as plsc

D = 64
SCALE = 8.0
NC = 2
NS = 16
NW = NC * NS
CHUNK = 128
NCHUNK = 200
PER_W = CHUNK * NCHUNK
NTOT = NW * PER_W
NBUF = 4
LANES = 16


def _scale_chunk(src, dst, b):
    @plsc.parallel_loop(0, CHUNK, unroll=8)
    def _(r):
        for c in range(D // LANES):
            sl = pl.ds(c * LANES, LANES)
            dst[b, r, sl] = src[b, r, sl] * SCALE


def _emb_body(x_hbm, tab_hbm, out_hbm, idx_v, gbuf, sbuf, gsem, ssem):
    cid = lax.axis_index("c")
    sid = lax.axis_index("s")
    wid = sid * NC + cid
    base = wid * PER_W

    pltpu.sync_copy(x_hbm.at[wid], idx_v)

    def gather_start(j, b):
        pltpu.async_copy(tab_hbm.at[idx_v.at[j]], gbuf.at[b], gsem.at[b])

    def gather_wait(j, b):
        pltpu.make_async_copy(tab_hbm.at[idx_v.at[j]], gbuf.at[b],
                              gsem.at[b]).wait()

    def scat_start(j, b):
        pltpu.async_copy(sbuf.at[b], out_hbm.at[pl.ds(base + j * CHUNK, CHUNK)],
                         ssem.at[b])

    def scat_wait(j, b):
        pltpu.make_async_copy(sbuf.at[b],
                              out_hbm.at[pl.ds(base + j * CHUNK, CHUNK)],
                              ssem.at[b]).wait()

    for b in range(NBUF):
        gather_start(b, b)

    for b in range(NBUF):
        gather_wait(b, b)
        _scale_chunk(gbuf, sbuf, b)
        scat_start(b, b)
        gather_start(NBUF + b, b)

    def round_body(g, carry):
        for b in range(NBUF):
            j = g * NBUF + b
            gather_wait(j, b)
            scat_wait(j - NBUF, b)
            _scale_chunk(gbuf, sbuf, b)
            scat_start(j, b)
            gather_start(j + NBUF, b)
        return carry

    lax.fori_loop(1, NCHUNK // NBUF - 1, round_body, 0)

    g_last = NCHUNK // NBUF - 1
    for b in range(NBUF):
        j = g_last * NBUF + b
        gather_wait(j, b)
        scat_wait(j - NBUF, b)
        _scale_chunk(gbuf, sbuf, b)
        scat_start(j, b)
    for b in range(NBUF):
        scat_wait(g_last * NBUF + b, b)


@jax.jit
def kernel(x, table):
    xi = x.reshape(NW, NCHUNK, CHUNK).astype(jnp.int32)
    mesh = plsc.VectorSubcoreMesh(core_axis_name="c", subcore_axis_name="s")
    out = pl.kernel(
        _emb_body,
        out_type=jax.ShapeDtypeStruct((NTOT, D), jnp.float32),
        mesh=mesh,
        scratch_types=[
            pltpu.VMEM((NCHUNK, CHUNK), jnp.int32),
            pltpu.VMEM((NBUF, CHUNK, D), jnp.float32),
            pltpu.VMEM((NBUF, CHUNK, D), jnp.float32),
            pltpu.SemaphoreType.DMA((NBUF,)),
            pltpu.SemaphoreType.DMA((NBUF,)),
        ],
        compiler_params=pltpu.CompilerParams(use_tc_tiling_on_sc=False),
    )(xi, table)
    return out.reshape(x.shape[0], x.shape[1], D)


# speedup vs baseline: 1.7983x; 1.7983x over previous
import jax
import jax.numpy as jnp
from jax import lax
from jax.experimental import pallas as pl
from jax.experimental.pallas import tpu as pltpu
from jax.experimental.pallas import tpu_sc as plsc

D = 64
SCALE = 8.0
NC = 2
NS = 16
NW = NC * NS
CHUNK = 128
NCH = 6400
PER_W = NCH // NW
NBUF = 4
LANES = 16
PITCH = 65


def _transpose_scale(gbuf, pbuf, sbuf, b):
    row_iota = lax.iota(jnp.int32, LANES)

    @plsc.parallel_loop(0, CHUNK * (D // LANES), unroll=8)
    def _(i):
        r = jnp.right_shift(i, 2)
        c = jnp.bitwise_and(i, 3) * LANES
        pbuf[r, pl.ds(c, LANES)] = gbuf[b, r, pl.ds(c, LANES)] * SCALE

    @plsc.parallel_loop(0, D * (CHUNK // LANES), unroll=8)
    def _(i):
        d = jnp.right_shift(i, 3)
        blk = jnp.bitwise_and(i, 7)
        dt = jnp.right_shift(d, 3)
        ds = jnp.bitwise_and(d, 7)
        rows = row_iota + blk * LANES
        cols = jnp.full((LANES,), d, jnp.int32)
        vals = plsc.load_gather(pbuf, [rows, cols])
        sbuf[b, dt, ds, pl.ds(blk * LANES, LANES)] = vals


def _emb_body(xi_hbm, tab_hbm, out_hbm, idx_v, gbuf, pbuf, sbuf, gsem, ssem):
    cid = lax.axis_index("c")
    sid = lax.axis_index("s")
    wid = sid * NC + cid
    c0 = wid * PER_W

    pltpu.sync_copy(xi_hbm.at[pl.ds(c0, PER_W)], idx_v)

    def gather_start(j, b):
        pltpu.async_copy(tab_hbm.at[idx_v.at[j]], gbuf.at[b], gsem.at[b])

    def gather_wait(j, b):
        pltpu.make_async_copy(tab_hbm.at[idx_v.at[j]], gbuf.at[b],
                              gsem.at[b]).wait()

    def out_ref(j):
        c = c0 + j
        h = jnp.right_shift(c, 7)
        bt = jnp.bitwise_and(c, 127)
        return out_hbm.at[h, :, bt]

    def scat_start(j, b):
        pltpu.async_copy(sbuf.at[b], out_ref(j), ssem.at[b])

    def scat_wait(j, b):
        pltpu.make_async_copy(sbuf.at[b], out_ref(j), ssem.at[b]).wait()

    for b in range(NBUF):
        gather_start(b, b)

    for b in range(NBUF):
        gather_wait(b, b)
        _transpose_scale(gbuf, pbuf, sbuf, b)
        scat_start(b, b)
        gather_start(NBUF + b, b)

    def round_body(g, carry):
        for b in range(NBUF):
            j = g * NBUF + b
            gather_wait(j, b)
            scat_wait(j - NBUF, b)
            _transpose_scale(gbuf, pbuf, sbuf, b)
            scat_start(j, b)
            gather_start(j + NBUF, b)
        return carry

    lax.fori_loop(1, PER_W // NBUF - 1, round_body, 0)

    g_last = PER_W // NBUF - 1
    for b in range(NBUF):
        j = g_last * NBUF + b
        gather_wait(j, b)
        scat_wait(j - NBUF, b)
        _transpose_scale(gbuf, pbuf, sbuf, b)
        scat_start(j, b)
    for b in range(NBUF):
        scat_wait(g_last * NBUF + b, b)


RELAY_BLK = 2048


def _detile_body(tt_ref, out_ref):
    a = tt_ref[...]
    b = a.T.reshape(RELAY_BLK, 2, D)
    out_ref[:, 0:D] = b[:, 0]
    out_ref[:, D:2 * D] = b[:, 1]


def _detile(tT):
    V = tT.shape[1]
    grid = (V // 2 + RELAY_BLK - 1) // RELAY_BLK
    return pl.pallas_call(
        _detile_body,
        grid=(grid,),
        in_specs=[pl.BlockSpec((D, 2 * RELAY_BLK), lambda j: (0, j))],
        out_specs=pl.BlockSpec((RELAY_BLK, 128), lambda j: (j, 0)),
        out_shape=jax.ShapeDtypeStruct((V // 2, 128), jnp.float32),
    )(tT)


@jax.jit
def kernel(x, table):
    B, H = x.shape
    V = table.shape[0]
    tab_lin = _detile(table.T).reshape(V, D)
    xi = jnp.transpose(x).reshape(NCH, CHUNK).astype(jnp.int32)
    mesh = plsc.VectorSubcoreMesh(core_axis_name="c", subcore_axis_name="s")
    out5 = pl.kernel(
        _emb_body,
        out_type=jax.ShapeDtypeStruct((H, D // 8, CHUNK, 8, CHUNK), jnp.float32),
        mesh=mesh,
        scratch_types=[
            pltpu.VMEM((PER_W, CHUNK), jnp.int32),
            pltpu.VMEM((NBUF, CHUNK, D), jnp.float32),
            pltpu.VMEM((CHUNK, PITCH), jnp.float32),
            pltpu.VMEM((NBUF, D // 8, 8, CHUNK), jnp.float32),
            pltpu.SemaphoreType.DMA((NBUF,)),
            pltpu.SemaphoreType.DMA((NBUF,)),
        ],
        compiler_params=pltpu.CompilerParams(use_tc_tiling_on_sc=False,
                                             needs_layout_passes=False),
    )(xi, tab_lin)
    return out5.transpose(2, 4, 0, 1, 3).reshape(B, H, D)


# speedup vs baseline: 2.0826x; 1.1581x over previous
import jax
import jax.numpy as jnp
from jax import lax
from jax.experimental import pallas as pl
from jax.experimental.pallas import tpu as pltpu
from jax.experimental.pallas import tpu_sc as plsc

D = 64
SCALE = 8.0
NC = 2
NS = 16
NW = NC * NS
CHUNK = 128
NCH = 6400
PER_W = NCH // NW
NBUF = 4
LANES = 16
PITCH = 65


def _transpose_scale(gbuf, pbuf, sbuf, b):
    row_iota = lax.iota(jnp.int32, LANES)

    @plsc.parallel_loop(0, CHUNK * (D // LANES), unroll=8)
    def _(i):
        r = jnp.right_shift(i, 2)
        c = jnp.bitwise_and(i, 3) * LANES
        pbuf[r, pl.ds(c, LANES)] = gbuf[b, r, pl.ds(c, LANES)] * SCALE

    @plsc.parallel_loop(0, D * (CHUNK // LANES), unroll=8)
    def _(i):
        d = jnp.right_shift(i, 3)
        blk = jnp.bitwise_and(i, 7)
        dt = jnp.right_shift(d, 3)
        ds = jnp.bitwise_and(d, 7)
        rows = row_iota + blk * LANES
        cols = jnp.full((LANES,), d, jnp.int32)
        vals = plsc.load_gather(pbuf, [rows, cols])
        sbuf[b, dt, ds, pl.ds(blk * LANES, LANES)] = vals


def _emb_body(xi_hbm, tab_hbm, out_hbm, idx_v, gbuf, pbuf, sbuf, gsem, ssem):
    cid = lax.axis_index("c")
    sid = lax.axis_index("s")
    wid = sid * NC + cid
    c0 = wid * PER_W

    pltpu.sync_copy(xi_hbm.at[pl.ds(c0, PER_W)], idx_v)

    def gather_start(j, b):
        pltpu.async_copy(tab_hbm.at[idx_v.at[j]], gbuf.at[b], gsem.at[b])

    def gather_wait(j, b):
        pltpu.make_async_copy(tab_hbm.at[idx_v.at[j]], gbuf.at[b],
                              gsem.at[b]).wait()

    def out_ref(j):
        c = c0 + j
        h = jnp.right_shift(c, 7)
        bt = jnp.bitwise_and(c, 127)
        return out_hbm.at[h, :, bt]

    def scat_start(j, b):
        pltpu.async_copy(sbuf.at[b], out_ref(j), ssem.at[b])

    def scat_wait(j, b):
        pltpu.make_async_copy(sbuf.at[b], out_ref(j), ssem.at[b]).wait()

    for b in range(NBUF):
        gather_start(b, b)

    for b in range(NBUF):
        gather_wait(b, b)
        _transpose_scale(gbuf, pbuf, sbuf, b)
        scat_start(b, b)
        gather_start(NBUF + b, b)

    def round_body(g, carry):
        for b in range(NBUF):
            j = g * NBUF + b
            gather_wait(j, b)
            scat_wait(j - NBUF, b)
            _transpose_scale(gbuf, pbuf, sbuf, b)
            scat_start(j, b)
            gather_start(j + NBUF, b)
        return carry

    lax.fori_loop(1, PER_W // NBUF - 1, round_body, 0)

    g_last = PER_W // NBUF - 1
    for b in range(NBUF):
        j = g_last * NBUF + b
        gather_wait(j, b)
        scat_wait(j - NBUF, b)
        _transpose_scale(gbuf, pbuf, sbuf, b)
        scat_start(j, b)
    for b in range(NBUF):
        scat_wait(g_last * NBUF + b, b)


RELAY_BLK = 2048


def _detile_body(tt_ref, out_ref):
    at = tt_ref[...].T
    out_ref[:, 0:D] = at[0:RELAY_BLK]
    out_ref[:, D:2 * D] = at[RELAY_BLK:2 * RELAY_BLK]


def _detile(tT):
    V = tT.shape[1]
    grid = (V // 2 + RELAY_BLK - 1) // RELAY_BLK
    return pl.pallas_call(
        _detile_body,
        grid=(grid,),
        in_specs=[pl.BlockSpec((D, 2 * RELAY_BLK), lambda j: (0, j))],
        out_specs=pl.BlockSpec((RELAY_BLK, 128), lambda j: (j, 0)),
        out_shape=jax.ShapeDtypeStruct((grid * RELAY_BLK, 128), jnp.float32),
    )(tT)


@jax.jit
def kernel(x, table):
    B, H = x.shape
    V = table.shape[0]
    lin = _detile(table.T)
    tab_lin = lin.reshape(2 * lin.shape[0], D)
    xr = x.astype(jnp.int32)
    TB = 2 * RELAY_BLK
    xv = (xr & ~(TB - 1)) + 2 * (xr & (RELAY_BLK - 1)) \
        + ((xr >> 11) & 1)
    xi = jnp.transpose(xv).reshape(NCH, CHUNK)
    mesh = plsc.VectorSubcoreMesh(core_axis_name="c", subcore_axis_name="s")
    out5 = pl.kernel(
        _emb_body,
        out_type=jax.ShapeDtypeStruct((H, D // 8, CHUNK, 8, CHUNK), jnp.float32),
        mesh=mesh,
        scratch_types=[
            pltpu.VMEM((PER_W, CHUNK), jnp.int32),
            pltpu.VMEM((NBUF, CHUNK, D), jnp.float32),
            pltpu.VMEM((CHUNK, PITCH), jnp.float32),
            pltpu.VMEM((NBUF, D // 8, 8, CHUNK), jnp.float32),
            pltpu.SemaphoreType.DMA((NBUF,)),
            pltpu.SemaphoreType.DMA((NBUF,)),
        ],
        compiler_params=pltpu.CompilerParams(use_tc_tiling_on_sc=False,
                                             needs_layout_passes=False),
    )(xi, tab_lin)
    return out5.transpose(2, 4, 0, 1, 3).reshape(B, H, D)


# speedup vs baseline: 2.3405x; 1.1239x over previous
import jax
import jax.numpy as jnp
from jax import lax
from jax.experimental import pallas as pl
from jax.experimental.pallas import tpu as pltpu
from jax.experimental.pallas import tpu_sc as plsc

D = 64
SCALE = 8.0
NC = 2
NS = 16
NW = NC * NS
CHUNK = 128
NCH = 6400
PER_W = NCH // NW
NBUF = 4
LANES = 16
PITCH = 65


def _transpose_scale(gbuf, pbuf, sbuf, b):
    row_iota = lax.iota(jnp.int32, LANES)

    @plsc.parallel_loop(0, CHUNK * (D // LANES), unroll=8)
    def _(i):
        r = jnp.right_shift(i, 2)
        c = jnp.bitwise_and(i, 3) * LANES
        pbuf[r, pl.ds(c, LANES)] = gbuf[b, r, pl.ds(c, LANES)] * SCALE

    @plsc.parallel_loop(0, D * (CHUNK // LANES), unroll=8)
    def _(i):
        d = jnp.right_shift(i, 3)
        blk = jnp.bitwise_and(i, 7)
        dt = jnp.right_shift(d, 3)
        ds = jnp.bitwise_and(d, 7)
        rows = row_iota + blk * LANES
        cols = jnp.full((LANES,), d, jnp.int32)
        vals = plsc.load_gather(pbuf, [rows, cols])
        sbuf[b, dt, ds, pl.ds(blk * LANES, LANES)] = vals


def _emb_body(xi_hbm, tab_hbm, out_hbm, idx_v, gbuf, pbuf, sbuf, gsem, ssem):
    cid = lax.axis_index("c")
    sid = lax.axis_index("s")
    wid = sid * NC + cid
    c0 = wid * PER_W

    pltpu.sync_copy(xi_hbm.at[pl.ds(c0, PER_W)], idx_v)

    def gather_start(j, b):
        pltpu.async_copy(tab_hbm.at[idx_v.at[j]], gbuf.at[b], gsem.at[b])

    def gather_wait(j, b):
        pltpu.make_async_copy(tab_hbm.at[idx_v.at[j]], gbuf.at[b],
                              gsem.at[b]).wait()

    def out_ref(j):
        c = c0 + j
        h = jnp.right_shift(c, 7)
        bt = jnp.bitwise_and(c, 127)
        return out_hbm.at[h, :, bt]

    def scat_start(j, b):
        pltpu.async_copy(sbuf.at[b], out_ref(j), ssem.at[b])

    def scat_wait(j, b):
        pltpu.make_async_copy(sbuf.at[b], out_ref(j), ssem.at[b]).wait()

    for b in range(NBUF):
        gather_start(b, b)

    for b in range(NBUF):
        gather_wait(b, b)
        _transpose_scale(gbuf, pbuf, sbuf, b)
        scat_start(b, b)
        gather_start(NBUF + b, b)

    def round_body(g, carry):
        for b in range(NBUF):
            j = g * NBUF + b
            gather_wait(j, b)
            scat_wait(j - NBUF, b)
            _transpose_scale(gbuf, pbuf, sbuf, b)
            scat_start(j, b)
            gather_start(j + NBUF, b)
        return carry

    lax.fori_loop(1, PER_W // NBUF - 1, round_body, 0)

    g_last = PER_W // NBUF - 1
    for b in range(NBUF):
        j = g_last * NBUF + b
        gather_wait(j, b)
        scat_wait(j - NBUF, b)
        _transpose_scale(gbuf, pbuf, sbuf, b)
        scat_start(j, b)
    for b in range(NBUF):
        scat_wait(g_last * NBUF + b, b)


RELAY_BLK = 4096


def _detile_body(tt_ref, out_ref):
    at = tt_ref[...].T
    out_ref[:, 0:D] = at[0:RELAY_BLK]
    out_ref[:, D:2 * D] = at[RELAY_BLK:2 * RELAY_BLK]


def _detile(tT):
    V = tT.shape[1]
    grid = (V // 2 + RELAY_BLK - 1) // RELAY_BLK
    return pl.pallas_call(
        _detile_body,
        grid=(grid,),
        in_specs=[pl.BlockSpec((D, 2 * RELAY_BLK), lambda j: (0, j))],
        out_specs=pl.BlockSpec((RELAY_BLK, 128), lambda j: (j, 0)),
        out_shape=jax.ShapeDtypeStruct((grid * RELAY_BLK, 128), jnp.float32),
    )(tT)


@jax.jit
def kernel(x, table):
    B, H = x.shape
    V = table.shape[0]
    lin = _detile(table.T)
    tab_lin = lin.reshape(2 * lin.shape[0], D)
    xr = x.astype(jnp.int32)
    TB = 2 * RELAY_BLK
    xv = (xr & ~(TB - 1)) + 2 * (xr & (RELAY_BLK - 1)) \
        + ((xr // RELAY_BLK) & 1)
    xi = jnp.transpose(xv).reshape(NCH, CHUNK)
    mesh = plsc.VectorSubcoreMesh(core_axis_name="c", subcore_axis_name="s")
    out5 = pl.kernel(
        _emb_body,
        out_type=jax.ShapeDtypeStruct((H, D // 8, CHUNK, 8, CHUNK), jnp.float32),
        mesh=mesh,
        scratch_types=[
            pltpu.VMEM((PER_W, CHUNK), jnp.int32),
            pltpu.VMEM((NBUF, CHUNK, D), jnp.float32),
            pltpu.VMEM((CHUNK, PITCH), jnp.float32),
            pltpu.VMEM((NBUF, D // 8, 8, CHUNK), jnp.float32),
            pltpu.SemaphoreType.DMA((NBUF,)),
            pltpu.SemaphoreType.DMA((NBUF,)),
        ],
        compiler_params=pltpu.CompilerParams(use_tc_tiling_on_sc=False,
                                             needs_layout_passes=False),
    )(xi, tab_lin)
    return out5.transpose(2, 4, 0, 1, 3).reshape(B, H, D)


# speedup vs baseline: 2.4776x; 1.0585x over previous
import jax
import jax.numpy as jnp
from jax import lax
from jax.experimental import pallas as pl
from jax.experimental.pallas import tpu as pltpu
from jax.experimental.pallas import tpu_sc as plsc

D = 64
SCALE = 8.0
NC = 2
NS = 16
NW = NC * NS
CHUNK = 128
NCH = 6400
PER_W = NCH // NW
NBUF = 4
LANES = 16
PITCH = 65


def _transpose_scale(gbuf, pbuf, sbuf, b):
    row_iota = lax.iota(jnp.int32, LANES)

    @plsc.parallel_loop(0, CHUNK * (D // LANES), unroll=8)
    def _(i):
        r = jnp.right_shift(i, 2)
        c = jnp.bitwise_and(i, 3) * LANES
        pbuf[r, pl.ds(c, LANES)] = gbuf[b, r, pl.ds(c, LANES)] * SCALE

    @plsc.parallel_loop(0, D * (CHUNK // LANES), unroll=8)
    def _(i):
        d = jnp.right_shift(i, 3)
        blk = jnp.bitwise_and(i, 7)
        dt = jnp.right_shift(d, 3)
        ds = jnp.bitwise_and(d, 7)
        rows = row_iota + blk * LANES
        cols = jnp.full((LANES,), d, jnp.int32)
        vals = plsc.load_gather(pbuf, [rows, cols])
        sbuf[b, dt, ds, pl.ds(blk * LANES, LANES)] = vals


def _emb_body(xi_hbm, tab_hbm, out_hbm, idx_v, gbuf, pbuf, sbuf, gsem, ssem):
    cid = lax.axis_index("c")
    sid = lax.axis_index("s")
    wid = sid * NC + cid
    c0 = wid * PER_W

    pltpu.sync_copy(xi_hbm.at[pl.ds(c0, PER_W)], idx_v)

    def gather_start(j, b):
        pltpu.async_copy(tab_hbm.at[idx_v.at[j]], gbuf.at[b], gsem.at[b])

    def gather_wait(j, b):
        pltpu.make_async_copy(tab_hbm.at[idx_v.at[j]], gbuf.at[b],
                              gsem.at[b]).wait()

    def out_ref(j):
        c = c0 + j
        h = jnp.right_shift(c, 7)
        bt = jnp.bitwise_and(c, 127)
        return out_hbm.at[h, :, bt]

    def scat_start(j, b):
        pltpu.async_copy(sbuf.at[b], out_ref(j), ssem.at[b])

    def scat_wait(j, b):
        pltpu.make_async_copy(sbuf.at[b], out_ref(j), ssem.at[b]).wait()

    for b in range(NBUF):
        gather_start(b, b)

    for b in range(NBUF):
        gather_wait(b, b)
        _transpose_scale(gbuf, pbuf, sbuf, b)
        scat_start(b, b)
        gather_start(NBUF + b, b)

    def round_body(g, carry):
        for b in range(NBUF):
            j = g * NBUF + b
            gather_wait(j, b)
            scat_wait(j - NBUF, b)
            _transpose_scale(gbuf, pbuf, sbuf, b)
            scat_start(j, b)
            gather_start(j + NBUF, b)
        return carry

    lax.fori_loop(1, PER_W // NBUF - 1, round_body, 0)

    g_last = PER_W // NBUF - 1
    for b in range(NBUF):
        j = g_last * NBUF + b
        gather_wait(j, b)
        scat_wait(j - NBUF, b)
        _transpose_scale(gbuf, pbuf, sbuf, b)
        scat_start(j, b)
    for b in range(NBUF):
        scat_wait(g_last * NBUF + b, b)


RELAY_BLK = 8192


def _detile_body(tt_ref, out_ref):
    at = tt_ref[...].T
    out_ref[:, 0:D] = at[0:RELAY_BLK]
    out_ref[:, D:2 * D] = at[RELAY_BLK:2 * RELAY_BLK]


def _detile(tT):
    V = tT.shape[1]
    grid = (V // 2 + RELAY_BLK - 1) // RELAY_BLK
    return pl.pallas_call(
        _detile_body,
        grid=(grid,),
        in_specs=[pl.BlockSpec((D, 2 * RELAY_BLK), lambda j: (0, j))],
        out_specs=pl.BlockSpec((RELAY_BLK, 128), lambda j: (j, 0)),
        out_shape=jax.ShapeDtypeStruct((grid * RELAY_BLK, 128), jnp.float32),
    )(tT)


@jax.jit
def kernel(x, table):
    B, H = x.shape
    V = table.shape[0]
    lin = _detile(table.T)
    tab_lin = lin.reshape(2 * lin.shape[0], D)
    xr = x.astype(jnp.int32)
    TB = 2 * RELAY_BLK
    xv = (xr & ~(TB - 1)) + 2 * (xr & (RELAY_BLK - 1)) \
        + ((xr // RELAY_BLK) & 1)
    xi = jnp.transpose(xv).reshape(NCH, CHUNK)
    mesh = plsc.VectorSubcoreMesh(core_axis_name="c", subcore_axis_name="s")
    out5 = pl.kernel(
        _emb_body,
        out_type=jax.ShapeDtypeStruct((H, D // 8, CHUNK, 8, CHUNK), jnp.float32),
        mesh=mesh,
        scratch_types=[
            pltpu.VMEM((PER_W, CHUNK), jnp.int32),
            pltpu.VMEM((NBUF, CHUNK, D), jnp.float32),
            pltpu.VMEM((CHUNK, PITCH), jnp.float32),
            pltpu.VMEM((NBUF, D // 8, 8, CHUNK), jnp.float32),
            pltpu.SemaphoreType.DMA((NBUF,)),
            pltpu.SemaphoreType.DMA((NBUF,)),
        ],
        compiler_params=pltpu.CompilerParams(use_tc_tiling_on_sc=False,
                                             needs_layout_passes=False),
    )(xi, tab_lin)
    return out5.transpose(2, 4, 0, 1, 3).reshape(B, H, D)


# speedup vs baseline: 2.5565x; 1.0318x over previous
import jax
import jax.numpy as jnp
from jax import lax
from jax.experimental import pallas as pl
from jax.experimental.pallas import tpu as pltpu
from jax.experimental.pallas import tpu_sc as plsc

D = 64
SCALE = 8.0
NC = 2
NS = 16
NW = NC * NS
CHUNK = 128
NCH = 6400
PER_W = NCH // NW
NBUF = 4
LANES = 16
PITCH = 65


def _transpose_scale(gbuf, pbuf, sbuf, b):
    row_iota = lax.iota(jnp.int32, LANES)

    @plsc.parallel_loop(0, CHUNK * (D // LANES), unroll=8)
    def _(i):
        r = jnp.right_shift(i, 2)
        c = jnp.bitwise_and(i, 3) * LANES
        pbuf[r, pl.ds(c, LANES)] = gbuf[b, r, pl.ds(c, LANES)] * SCALE

    @plsc.parallel_loop(0, D * (CHUNK // LANES), unroll=8)
    def _(i):
        d = jnp.right_shift(i, 3)
        blk = jnp.bitwise_and(i, 7)
        dt = jnp.right_shift(d, 3)
        ds = jnp.bitwise_and(d, 7)
        rows = row_iota + blk * LANES
        cols = jnp.full((LANES,), d, jnp.int32)
        vals = plsc.load_gather(pbuf, [rows, cols])
        sbuf[b, dt, ds, pl.ds(blk * LANES, LANES)] = vals


def _emb_body(xi_hbm, tab_hbm, out_hbm, idx_v, gbuf, pbuf, sbuf, gsem, ssem):
    cid = lax.axis_index("c")
    sid = lax.axis_index("s")
    wid = sid * NC + cid
    c0 = wid * PER_W

    pltpu.sync_copy(xi_hbm.at[pl.ds(c0, PER_W)], idx_v)

    def gather_start(j, b):
        pltpu.async_copy(tab_hbm.at[idx_v.at[j]], gbuf.at[b], gsem.at[b])

    def gather_wait(j, b):
        pltpu.make_async_copy(tab_hbm.at[idx_v.at[j]], gbuf.at[b],
                              gsem.at[b]).wait()

    def out_ref(j):
        c = c0 + j
        h = jnp.right_shift(c, 7)
        bt = jnp.bitwise_and(c, 127)
        return out_hbm.at[h, :, bt]

    def scat_start(j, b):
        pltpu.async_copy(sbuf.at[b], out_ref(j), ssem.at[b])

    def scat_wait(j, b):
        pltpu.make_async_copy(sbuf.at[b], out_ref(j), ssem.at[b]).wait()

    for b in range(NBUF):
        gather_start(b, b)

    for b in range(NBUF):
        gather_wait(b, b)
        _transpose_scale(gbuf, pbuf, sbuf, b)
        scat_start(b, b)
        gather_start(NBUF + b, b)

    def round_body(g, carry):
        for b in range(NBUF):
            j = g * NBUF + b
            gather_wait(j, b)
            scat_wait(j - NBUF, b)
            _transpose_scale(gbuf, pbuf, sbuf, b)
            scat_start(j, b)
            gather_start(j + NBUF, b)
        return carry

    lax.fori_loop(1, PER_W // NBUF - 1, round_body, 0)

    g_last = PER_W // NBUF - 1
    for b in range(NBUF):
        j = g_last * NBUF + b
        gather_wait(j, b)
        scat_wait(j - NBUF, b)
        _transpose_scale(gbuf, pbuf, sbuf, b)
        scat_start(j, b)
    for b in range(NBUF):
        scat_wait(g_last * NBUF + b, b)


RELAY_BLK = 16384


def _detile_body(tt_ref, out_ref):
    at = tt_ref[...].T
    out_ref[:, 0:D] = at[0:RELAY_BLK]
    out_ref[:, D:2 * D] = at[RELAY_BLK:2 * RELAY_BLK]


def _detile(tT):
    V = tT.shape[1]
    grid = (V // 2 + RELAY_BLK - 1) // RELAY_BLK
    return pl.pallas_call(
        _detile_body,
        grid=(grid,),
        in_specs=[pl.BlockSpec((D, 2 * RELAY_BLK), lambda j: (0, j))],
        out_specs=pl.BlockSpec((RELAY_BLK, 128), lambda j: (j, 0)),
        out_shape=jax.ShapeDtypeStruct((grid * RELAY_BLK, 128), jnp.float32),
    )(tT)


@jax.jit
def kernel(x, table):
    B, H = x.shape
    V = table.shape[0]
    lin = _detile(table.T)
    tab_lin = lin.reshape(2 * lin.shape[0], D)
    xr = x.astype(jnp.int32)
    TB = 2 * RELAY_BLK
    xv = (xr & ~(TB - 1)) + 2 * (xr & (RELAY_BLK - 1)) \
        + ((xr // RELAY_BLK) & 1)
    xi = jnp.transpose(xv).reshape(NCH, CHUNK)
    mesh = plsc.VectorSubcoreMesh(core_axis_name="c", subcore_axis_name="s")
    out5 = pl.kernel(
        _emb_body,
        out_type=jax.ShapeDtypeStruct((H, D // 8, CHUNK, 8, CHUNK), jnp.float32),
        mesh=mesh,
        scratch_types=[
            pltpu.VMEM((PER_W, CHUNK), jnp.int32),
            pltpu.VMEM((NBUF, CHUNK, D), jnp.float32),
            pltpu.VMEM((CHUNK, PITCH), jnp.float32),
            pltpu.VMEM((NBUF, D // 8, 8, CHUNK), jnp.float32),
            pltpu.SemaphoreType.DMA((NBUF,)),
            pltpu.SemaphoreType.DMA((NBUF,)),
        ],
        compiler_params=pltpu.CompilerParams(use_tc_tiling_on_sc=False,
                                             needs_layout_passes=False),
    )(xi, tab_lin)
    return out5.transpose(2, 4, 0, 1, 3).reshape(B, H, D)
